# baseline (speedup 1.0000x reference)
import functools

import jax
import jax.numpy as jnp
from jax import lax
from jax.experimental import pallas as pl
from jax.experimental.pallas import tpu as pltpu
from jax.experimental.pallas import tpu_sc as plsc

D = 256
T = 512
B = 8
MAX_MEL = 2000
M_PAD = 2048
N_BINS = 256
EMB_REP = 64
_EPS = 1e-5


def _ln(v, g, b):
    mu = jnp.mean(v, axis=-1, keepdims=True)
    var = jnp.mean((v - mu) ** 2, axis=-1, keepdims=True)
    return (v - mu) / jnp.sqrt(var + _EPS) * g + b


def _fused_body(x_ref, maskf_ref, w1_ref, b1_ref, g1_ref, bb1_ref,
                w2_ref, b2_ref, g2_ref, bb2_ref, wl_ref, bl_ref,
                ptab_ref, etab_ref,
                table_ref, logd_ref, pitch_ref, energy_ref, dur_ref,
                g_ref, melmask_ref, mellen_ref):
    x = x_ref[...]
    maskf = maskf_ref[...]
    n = B * T
    rows = lax.broadcasted_iota(jnp.int32, (n, 1), 0)
    tmod = rows & (T - 1)
    first_row = tmod == 0
    last_row = tmod == (T - 1)
    z = jnp.zeros((1, D), jnp.float32)

    def conv3(v, wref, bref, p):
        down = jnp.concatenate([z, v[:n - 1]], axis=0)
        up = jnp.concatenate([v[1:], z], axis=0)
        down = jnp.where(first_row, 0.0, down)
        up = jnp.where(last_row, 0.0, up)
        acc = jnp.dot(down, wref[p, 0], preferred_element_type=jnp.float32)
        acc += jnp.dot(v, wref[p, 1], preferred_element_type=jnp.float32)
        acc += jnp.dot(up, wref[p, 2], preferred_element_type=jnp.float32)
        return acc + bref[p]

    outs = []
    for p in range(3):
        h = jax.nn.relu(conv3(x, w1_ref, b1_ref, p))
        h = _ln(h, g1_ref[p], bb1_ref[p])
        h = jax.nn.relu(conv3(h, w2_ref, b2_ref, p))
        h = _ln(h, g2_ref[p], bb2_ref[p])
        o = jnp.dot(h, wl_ref[p], preferred_element_type=jnp.float32)
        o = (o + bl_ref[p]) * (1.0 - maskf)
        outs.append(o)
    logd_ref[...] = outs[0]
    pitch_ref[...] = outs[1]
    energy_ref[...] = outs[2]

    dur_f = jnp.maximum(jnp.round(jnp.exp(outs[0]) - 1.0), 0.0)
    dur_ref[...] = dur_f.astype(jnp.int32)

    it = lax.broadcasted_iota(jnp.int32, (T, T), 0)
    js = lax.broadcasted_iota(jnp.int32, (T, T), 1)
    tri = (js <= it).astype(jnp.float32)
    frames = lax.broadcasted_iota(jnp.int32, (1, M_PAD), 1).astype(jnp.float32)
    mrep = lax.broadcasted_iota(jnp.int32, (1, M_PAD), 1) & (EMB_REP - 1)
    lanes = lax.broadcasted_iota(jnp.int32, (1, N_BINS), 1)
    ptab = ptab_ref[...]
    etab = etab_ref[...]

    for b in range(B):
        sl = slice(b * T, (b + 1) * T)
        cum = jnp.dot(tri, dur_f[sl], preferred_element_type=jnp.float32)
        cum_last = cum[T - 1:T, :]
        mel_len = jnp.minimum(cum_last, float(MAX_MEL))
        mellen_ref[b:b + 1, :] = mel_len.astype(jnp.int32)

        cmp = (cum <= frames).astype(jnp.float32)
        tid = jnp.sum(cmp, axis=0, keepdims=True)
        valid = frames < mel_len
        g = jnp.where(valid, tid + float(b * T),
                      float(B * T + b * EMB_REP) + mrep.astype(jnp.float32))
        gi = g.astype(jnp.int32)
        for q in range(4):
            g_ref[4 * b + q:4 * b + q + 1, :] = (
                gi[:, q * _W_OFF:q * _W_OFF + _IDX_PAD])
        melmask_ref[b:b + 1, :] = (frames >= mel_len).astype(
            jnp.int32)[:, :MAX_MEL]

        p_idx = jnp.clip(jnp.mean(outs[1][sl]).astype(jnp.int32), 0, N_BINS - 1)
        e_idx = jnp.clip(jnp.mean(outs[2][sl]).astype(jnp.int32), 0, N_BINS - 1)
        oh_p = (lanes == p_idx).astype(jnp.float32)
        oh_e = (lanes == e_idx).astype(jnp.float32)
        emb = (jnp.dot(oh_p, ptab, preferred_element_type=jnp.float32)
               + jnp.dot(oh_e, etab, preferred_element_type=jnp.float32))
        table_ref[sl, :] = x[sl] + emb
        table_ref[n + b * EMB_REP:n + (b + 1) * EMB_REP, :] = (
            jnp.broadcast_to(emb, (EMB_REP, D)))


def _fused(x2d, maskf2d, stk, ptab, etab):
    n = B * T
    return pl.pallas_call(
        _fused_body,
        out_shape=[
            jax.ShapeDtypeStruct((n + B * EMB_REP, D), jnp.float32),
            jax.ShapeDtypeStruct((n, 1), jnp.float32),
            jax.ShapeDtypeStruct((n, 1), jnp.float32),
            jax.ShapeDtypeStruct((n, 1), jnp.float32),
            jax.ShapeDtypeStruct((n, 1), jnp.int32),
            jax.ShapeDtypeStruct((32, _IDX_PAD), jnp.int32),
            jax.ShapeDtypeStruct((B, MAX_MEL), jnp.int32),
            jax.ShapeDtypeStruct((B, 1), jnp.int32),
        ],
    )(x2d, maskf2d, stk['w1'], stk['b1'], stk['g1'], stk['bb1'],
      stk['w2'], stk['b2'], stk['g2'], stk['bb2'], stk['wl'], stk['bl'],
      ptab, etab)


_N_OUT = B * MAX_MEL
_W_OFF = 496
_IDX_PAD = 512
_CHUNK = 64
_NCHUNK = _IDX_PAD // _CHUNK


def _sc_gather(table2, gpad):
    mesh = plsc.VectorSubcoreMesh(core_axis_name="c", subcore_axis_name="s")

    @functools.partial(
        pl.kernel,
        mesh=mesh,
        out_type=jax.ShapeDtypeStruct((_N_OUT, D), jnp.float32),
        scratch_types=[
            pltpu.VMEM((_IDX_PAD,), jnp.int32),
            pltpu.VMEM((_CHUNK, D), jnp.float32),
            pltpu.VMEM((_CHUNK, D), jnp.float32),
            pltpu.SemaphoreType.DMA,
            pltpu.SemaphoreType.DMA,
            pltpu.SemaphoreType.DMA,
            pltpu.SemaphoreType.DMA,
        ],
    )
    def k(table_hbm, idx_hbm, out_hbm, idx_v, buf0, buf1,
          gsem0, gsem1, wsem0, wsem1):
        wid = lax.axis_index("s") * 2 + lax.axis_index("c")
        base = (wid >> 2) * MAX_MEL + (wid & 3) * _W_OFF
        pltpu.sync_copy(idx_hbm.at[wid], idx_v)
        bufs = (buf0, buf1)
        gsems = (gsem0, gsem1)
        wsems = (wsem0, wsem1)
        gps = [None, None]
        wps = [None, None]
        gps[0] = pltpu.async_copy(
            table_hbm.at[idx_v.at[pl.ds(0, _CHUNK)]], bufs[0], gsems[0])
        for c in range(1, _NCHUNK):
            s = c & 1
            p = (c - 1) & 1
            if wps[s] is not None:
                wps[s].wait()
            gps[s] = pltpu.async_copy(
                table_hbm.at[idx_v.at[pl.ds(c * _CHUNK, _CHUNK)]],
                bufs[s], gsems[s])
            gps[p].wait()
            wps[p] = pltpu.async_copy(
                bufs[p], out_hbm.at[pl.ds(base + (c - 1) * _CHUNK, _CHUNK)],
                wsems[p])
        last = (_NCHUNK - 1) & 1
        gps[last].wait()
        wps[last] = pltpu.async_copy(
            bufs[last],
            out_hbm.at[pl.ds(base + (_NCHUNK - 1) * _CHUNK, _CHUNK)],
            wsems[last])
        wps[0].wait()
        wps[1].wait()

    return k(table2, gpad)


def kernel(x, mask, dur_params, pitch_params, energy_params, pitch_table, energy_table):
    maskf2d = mask.astype(jnp.float32).reshape(B * T, 1)
    stk = {k: jnp.stack([dur_params[k], pitch_params[k], energy_params[k]])
           for k in dur_params}
    for k in ('b1', 'g1', 'bb1', 'b2', 'g2', 'bb2'):
        stk[k] = stk[k].reshape(3, 1, D)
    stk['bl'] = stk['bl'].reshape(3, 1, 1)

    table, logd, pitch, energy, dur_i, g, melmask_i, mellen_i = _fused(
        x.reshape(B * T, D), maskf2d, stk, pitch_table, energy_table)

    out = _sc_gather(table, g)
    xe = out.reshape(B, MAX_MEL, D)

    return (xe, pitch.reshape(B, T), energy.reshape(B, T),
            logd.reshape(B, T), dur_i.reshape(B, T),
            mellen_i.reshape(B), melmask_i.astype(bool))


# speedup vs baseline: 1.1772x; 1.1772x over previous
import functools

import jax
import jax.numpy as jnp
from jax import lax
from jax.experimental import pallas as pl
from jax.experimental.pallas import tpu as pltpu
from jax.experimental.pallas import tpu_sc as plsc

D = 256
T = 512
B = 8
MAX_MEL = 2000
M_PAD = 2048
N_BINS = 256
EMB_REP = 64
_EPS = 1e-5


def _fused_body(x_ref, maskf_ref, *refs):
    (table_ref, logd_ref, pitch_ref, energy_ref, dur_ref,
     g_ref, melmask_ref, mellen_ref) = refs[32:]
    ptab_ref, etab_ref = refs[30], refs[31]
    x = x_ref[...]
    maskf = maskf_ref[...]
    n = B * T
    rows = lax.broadcasted_iota(jnp.int32, (n, 1), 0)
    tmod = rows & (T - 1)
    first_row = tmod == 0
    last_row = tmod == (T - 1)
    z = jnp.zeros((1, D), jnp.float32)
    ones_col = jnp.ones((D, 1), jnp.float32)

    def ln(v, g, b):
        mu = jnp.mean(v, axis=-1, keepdims=True)
        c = v - mu
        var = jnp.mean(c * c, axis=-1, keepdims=True)
        return c * lax.rsqrt(var + _EPS) * g + b

    def conv3(v, w3_ref, b):
        down = jnp.concatenate([z, v[:n - 1]], axis=0)
        up = jnp.concatenate([v[1:], z], axis=0)
        down = jnp.where(first_row, 0.0, down)
        up = jnp.where(last_row, 0.0, up)
        acc = jnp.dot(down, w3_ref[0], preferred_element_type=jnp.float32)
        acc += jnp.dot(v, w3_ref[1], preferred_element_type=jnp.float32)
        acc += jnp.dot(up, w3_ref[2], preferred_element_type=jnp.float32)
        return acc + b

    outs = []
    for p in range(3):
        (w1, b1, g1, bb1, w2, b2, g2, bb2, wl, bl) = refs[10 * p:10 * (p + 1)]
        h = jax.nn.relu(conv3(x, w1, b1[...]))
        h = ln(h, g1[...], bb1[...])
        h = jax.nn.relu(conv3(h, w2, b2[...]))
        h = ln(h, g2[...], bb2[...])
        o = jnp.dot(h, wl[...], preferred_element_type=jnp.float32)
        o = (o + bl[...]) * (1.0 - maskf)
        outs.append(o)
    logd_ref[...] = outs[0]
    pitch_ref[...] = outs[1]
    energy_ref[...] = outs[2]

    dur_f = jnp.maximum(jnp.round(jnp.exp(outs[0]) - 1.0), 0.0)
    dur_ref[...] = dur_f.astype(jnp.int32)

    it = lax.broadcasted_iota(jnp.int32, (T, T), 0)
    js = lax.broadcasted_iota(jnp.int32, (T, T), 1)
    tri = (js <= it).astype(jnp.float32)
    frames = lax.broadcasted_iota(jnp.int32, (1, M_PAD), 1).astype(jnp.float32)
    mrep = lax.broadcasted_iota(jnp.int32, (1, M_PAD), 1) & (EMB_REP - 1)
    lanes = lax.broadcasted_iota(jnp.int32, (1, N_BINS), 1)
    ones_row = jnp.ones((1, T), jnp.float32)
    ptab = ptab_ref[...]
    etab = etab_ref[...]

    for b in range(B):
        sl = slice(b * T, (b + 1) * T)
        cum = jnp.dot(tri, dur_f[sl], preferred_element_type=jnp.float32)
        cum_last = cum[T - 1:T, :]
        mel_len = jnp.minimum(cum_last, float(MAX_MEL))
        mellen_ref[b:b + 1, :] = mel_len.astype(jnp.int32)

        cmp = (cum <= frames).astype(jnp.float32)
        tid = jnp.dot(ones_row, cmp, preferred_element_type=jnp.float32)
        valid = frames < mel_len
        g = jnp.where(valid, tid + float(b * T),
                      float(B * T + b * EMB_REP) + mrep.astype(jnp.float32))
        gi = g.astype(jnp.int32)
        for q in range(4):
            g_ref[4 * b + q:4 * b + q + 1, :] = (
                gi[:, q * _W_OFF:q * _W_OFF + _IDX_PAD])
        melmask_ref[b:b + 1, :] = (frames >= mel_len).astype(
            jnp.int32)[:, :MAX_MEL]

        p_idx = jnp.clip(jnp.mean(outs[1][sl]).astype(jnp.int32), 0, N_BINS - 1)
        e_idx = jnp.clip(jnp.mean(outs[2][sl]).astype(jnp.int32), 0, N_BINS - 1)
        oh_p = (lanes == p_idx).astype(jnp.float32)
        oh_e = (lanes == e_idx).astype(jnp.float32)
        emb = (jnp.dot(oh_p, ptab, preferred_element_type=jnp.float32)
               + jnp.dot(oh_e, etab, preferred_element_type=jnp.float32))
        table_ref[sl, :] = x[sl] + emb
        table_ref[n + b * EMB_REP:n + (b + 1) * EMB_REP, :] = (
            jnp.broadcast_to(emb, (EMB_REP, D)))


_PKEYS = ('w1', 'b1', 'g1', 'bb1', 'w2', 'b2', 'g2', 'bb2', 'wl', 'bl')


def _fused(x2d, maskf2d, pflat, ptab, etab):
    n = B * T
    return pl.pallas_call(
        _fused_body,
        out_shape=[
            jax.ShapeDtypeStruct((n + B * EMB_REP, D), jnp.float32),
            jax.ShapeDtypeStruct((n, 1), jnp.float32),
            jax.ShapeDtypeStruct((n, 1), jnp.float32),
            jax.ShapeDtypeStruct((n, 1), jnp.float32),
            jax.ShapeDtypeStruct((n, 1), jnp.int32),
            jax.ShapeDtypeStruct((32, _IDX_PAD), jnp.int32),
            jax.ShapeDtypeStruct((B, MAX_MEL), jnp.int32),
            jax.ShapeDtypeStruct((B, 1), jnp.int32),
        ],
    )(x2d, maskf2d, *pflat, ptab, etab)


_N_OUT = B * MAX_MEL
_W_OFF = 496
_IDX_PAD = 512
_CHUNK = 64
_NCHUNK = _IDX_PAD // _CHUNK


def _sc_gather(table2, gpad):
    mesh = plsc.VectorSubcoreMesh(core_axis_name="c", subcore_axis_name="s")

    @functools.partial(
        pl.kernel,
        mesh=mesh,
        out_type=jax.ShapeDtypeStruct((_N_OUT, D), jnp.float32),
        scratch_types=[
            pltpu.VMEM((_IDX_PAD,), jnp.int32),
            pltpu.VMEM((_CHUNK, D), jnp.float32),
            pltpu.VMEM((_CHUNK, D), jnp.float32),
            pltpu.SemaphoreType.DMA,
            pltpu.SemaphoreType.DMA,
            pltpu.SemaphoreType.DMA,
            pltpu.SemaphoreType.DMA,
        ],
    )
    def k(table_hbm, idx_hbm, out_hbm, idx_v, buf0, buf1,
          gsem0, gsem1, wsem0, wsem1):
        wid = lax.axis_index("s") * 2 + lax.axis_index("c")
        base = (wid >> 2) * MAX_MEL + (wid & 3) * _W_OFF
        pltpu.sync_copy(idx_hbm.at[wid], idx_v)
        bufs = (buf0, buf1)
        gsems = (gsem0, gsem1)
        wsems = (wsem0, wsem1)
        gps = [None, None]
        wps = [None, None]
        gps[0] = pltpu.async_copy(
            table_hbm.at[idx_v.at[pl.ds(0, _CHUNK)]], bufs[0], gsems[0])
        for c in range(1, _NCHUNK):
            s = c & 1
            p = (c - 1) & 1
            if wps[s] is not None:
                wps[s].wait()
            gps[s] = pltpu.async_copy(
                table_hbm.at[idx_v.at[pl.ds(c * _CHUNK, _CHUNK)]],
                bufs[s], gsems[s])
            gps[p].wait()
            wps[p] = pltpu.async_copy(
                bufs[p], out_hbm.at[pl.ds(base + (c - 1) * _CHUNK, _CHUNK)],
                wsems[p])
        last = (_NCHUNK - 1) & 1
        gps[last].wait()
        wps[last] = pltpu.async_copy(
            bufs[last],
            out_hbm.at[pl.ds(base + (_NCHUNK - 1) * _CHUNK, _CHUNK)],
            wsems[last])
        wps[0].wait()
        wps[1].wait()

    return k(table2, gpad)


def kernel(x, mask, dur_params, pitch_params, energy_params, pitch_table, energy_table):
    maskf2d = mask.astype(jnp.float32).reshape(B * T, 1)
    pflat = []
    for prm in (dur_params, pitch_params, energy_params):
        for k in _PKEYS:
            v = prm[k]
            if k == 'bl':
                v = v.reshape(1, 1)
            elif v.ndim == 1:
                v = v.reshape(1, D)
            pflat.append(v)

    table, logd, pitch, energy, dur_i, g, melmask_i, mellen_i = _fused(
        x.reshape(B * T, D), maskf2d, pflat, pitch_table, energy_table)

    out = _sc_gather(table, g)
    xe = out.reshape(B, MAX_MEL, D)

    return (xe, pitch.reshape(B, T), energy.reshape(B, T),
            logd.reshape(B, T), dur_i.reshape(B, T),
            mellen_i.reshape(B), melmask_i.astype(bool))
